# baseline (speedup 1.0000x reference)
import functools

import jax
import jax.numpy as jnp
from jax import lax
from jax.experimental import pallas as pl
from jax.experimental.pallas import tpu as pltpu
from jax.experimental.pallas import tpu_sc as plsc

B = 16384
D = 32
L = 16
NC, NS = 2, 16
NW = NC * NS
BPW = B // NW
GROUPS = BPW // L

_MESH = plsc.VectorSubcoreMesh(core_axis_name="c", subcore_axis_name="s")


@functools.partial(
    pl.kernel,
    out_type=jax.ShapeDtypeStruct((B,), jnp.float32),
    mesh=_MESH,
    scratch_types=[
        pltpu.VMEM((BPW,), jnp.int32),
        pltpu.VMEM((BPW,), jnp.int32),
        pltpu.VMEM((BPW, D), jnp.float32),
        pltpu.VMEM((BPW, D), jnp.float32),
        pltpu.VMEM((BPW,), jnp.float32),
        pltpu.VMEM((BPW,), jnp.float32),
        pltpu.VMEM((BPW,), jnp.float32),
        pltpu.SemaphoreType.DMA,
    ],
    compiler_params=pltpu.CompilerParams(needs_layout_passes=False,
                                         use_tc_tiling_on_sc=False),
)
def _mf_sc(uids_hbm, iids_hbm, uemb_hbm, vemb_hbm, ubias_hbm, ibias_hbm,
           out_hbm, idx_u, idx_i, urows, vrows, ub_v, ib_v, out_v, sem):
    wid = lax.axis_index("s") * NC + lax.axis_index("c")
    base = wid * BPW

    pltpu.sync_copy(uids_hbm.at[pl.ds(base, BPW)], idx_u)
    pltpu.sync_copy(iids_hbm.at[pl.ds(base, BPW)], idx_i)

    cp1 = pltpu.async_copy(uemb_hbm.at[idx_u], urows, sem)
    cp2 = pltpu.async_copy(vemb_hbm.at[idx_i], vrows, sem)
    cp3 = pltpu.async_copy(ubias_hbm.at[idx_u], ub_v, sem)
    cp4 = pltpu.async_copy(ibias_hbm.at[idx_i], ib_v, sem)
    cp1.wait()
    cp2.wait()
    cp3.wait()
    cp4.wait()

    iota16 = lax.iota(jnp.int32, L)

    def group(g, carry):
        jvec = g * L + iota16
        acc = ub_v[pl.ds(g * L, L)] + ib_v[pl.ds(g * L, L)]
        for d in range(D):
            dvec = jnp.full((L,), d, jnp.int32)
            acc = acc + plsc.load_gather(urows, [jvec, dvec]) * plsc.load_gather(
                vrows, [jvec, dvec])
        out_v[pl.ds(g * L, L)] = acc
        return carry

    lax.fori_loop(0, GROUPS, group, 0)
    pltpu.sync_copy(out_v, out_hbm.at[pl.ds(base, BPW)])


def kernel(user_ids, item_ids, user_emb, item_emb, user_bias, item_bias):
    return _mf_sc(user_ids.astype(jnp.int32), item_ids.astype(jnp.int32),
                  user_emb, item_emb,
                  user_bias.reshape(-1), item_bias.reshape(-1))
